# baseline (speedup 1.0000x reference)
import functools

import jax
import jax.numpy as jnp
from jax.experimental import pallas as pl
from jax.experimental.pallas import tpu as pltpu


def _fused_body(n_agents, action_dim, gb, bb_rows, ti,
                actions_ref, states_ref, ww_ref, bw_ref, wb_ref, bb_ref,
                out_ref, mixed_s, b_s):
    N, A = n_agents, action_dim
    t = pl.program_id(0)

    @pl.when(t < gb)
    def _hypermix():
        states = states_ref[...]
        hw = jnp.dot(states, ww_ref[...],
                     preferred_element_type=jnp.float32) + bw_ref[...]
        b = jnp.dot(states, wb_ref[...],
                    preferred_element_type=jnp.float32) + bb_ref[...]
        acts = actions_ref[...]
        mixed = acts[:, 0:1] * jnp.abs(hw[:, 0:A])
        for n in range(1, N):
            mixed = mixed + acts[:, n:n + 1] * jnp.abs(hw[:, n * A:(n + 1) * A])
        row0 = t * bb_rows
        mixed_s[pl.ds(row0, bb_rows), :] = mixed
        b_s[pl.ds(row0, bb_rows), :] = b

    @pl.when(t >= gb)
    def _broadcast():
        rows = b_s[pl.ds((t - gb) * ti, ti), :]
        out_ref[...] = rows[:, None, :] + mixed_s[...][None, :, :]


def kernel(actions, states, ww, bw, wb, bb):
    f32 = jnp.float32
    actions = jnp.asarray(actions, f32)
    states = jnp.asarray(states, f32)
    B, N = actions.shape
    S = states.shape[1]
    NA = ww.shape[1]
    A = wb.shape[1]
    assert NA == N * A
    assert B % 64 == 0

    BB = B // 4
    TI = 8
    GB, GI = B // BB, B // TI

    out = pl.pallas_call(
        functools.partial(_fused_body, N, A, GB, BB, TI),
        grid=(GB + GI,),
        in_specs=[
            pl.BlockSpec((BB, N), lambda t: (jnp.minimum(t, 3), 0)),
            pl.BlockSpec((BB, S), lambda t: (jnp.minimum(t, 3), 0)),
            pl.BlockSpec((S, NA), lambda t: (0, 0)),
            pl.BlockSpec((1, NA), lambda t: (0, 0)),
            pl.BlockSpec((S, A), lambda t: (0, 0)),
            pl.BlockSpec((1, A), lambda t: (0, 0)),
        ],
        out_specs=pl.BlockSpec(
            (TI, B, A), lambda t: (jnp.maximum(t - GB, 0), 0, 0)),
        out_shape=jax.ShapeDtypeStruct((B, B, A), f32),
        scratch_shapes=[pltpu.VMEM((B, A), f32),
                        pltpu.VMEM((B, A), f32)],
        compiler_params=pltpu.CompilerParams(
            dimension_semantics=("arbitrary",)),
    )(actions, states, ww.astype(f32), bw.astype(f32),
      wb.astype(f32), bb.astype(f32))
    return out


# speedup vs baseline: 1.0021x; 1.0021x over previous
import functools

import jax
import jax.numpy as jnp
from jax.experimental import pallas as pl
from jax.experimental.pallas import tpu as pltpu


def _fused_body(n_agents, action_dim, gb, bb_rows, ti,
                actions_ref, states_ref, ww_ref, bw_ref, wb_ref, bb_ref,
                out_ref, mixed_s, b_s):
    N, A = n_agents, action_dim
    t = pl.program_id(0)

    @pl.when(t < gb)
    def _hypermix():
        states = states_ref[...]
        hw = jnp.dot(states, ww_ref[...],
                     preferred_element_type=jnp.float32) + bw_ref[...]
        b = jnp.dot(states, wb_ref[...],
                    preferred_element_type=jnp.float32) + bb_ref[...]
        acts = actions_ref[...]
        mixed = acts[:, 0:1] * jnp.abs(hw[:, 0:A])
        for n in range(1, N):
            mixed = mixed + acts[:, n:n + 1] * jnp.abs(hw[:, n * A:(n + 1) * A])
        row0 = t * bb_rows
        mixed_s[pl.ds(row0, bb_rows), :] = mixed
        b_s[pl.ds(row0, bb_rows), :] = b

    @pl.when(t >= gb)
    def _broadcast():
        rows = b_s[pl.ds((t - gb) * ti, ti), :]
        out_ref[...] = rows[:, None, :] + mixed_s[...][None, :, :]


def kernel(actions, states, ww, bw, wb, bb):
    f32 = jnp.float32
    actions = jnp.asarray(actions, f32)
    states = jnp.asarray(states, f32)
    B, N = actions.shape
    S = states.shape[1]
    NA = ww.shape[1]
    A = wb.shape[1]
    assert NA == N * A
    assert B % 64 == 0

    BB = B // 4
    TI = 16
    GB, GI = B // BB, B // TI

    out = pl.pallas_call(
        functools.partial(_fused_body, N, A, GB, BB, TI),
        grid=(GB + GI,),
        in_specs=[
            pl.BlockSpec((BB, N), lambda t: (jnp.minimum(t, 3), 0)),
            pl.BlockSpec((BB, S), lambda t: (jnp.minimum(t, 3), 0)),
            pl.BlockSpec((S, NA), lambda t: (0, 0)),
            pl.BlockSpec((1, NA), lambda t: (0, 0)),
            pl.BlockSpec((S, A), lambda t: (0, 0)),
            pl.BlockSpec((1, A), lambda t: (0, 0)),
        ],
        out_specs=pl.BlockSpec(
            (TI, B, A), lambda t: (jnp.maximum(t - GB, 0), 0, 0)),
        out_shape=jax.ShapeDtypeStruct((B, B, A), f32),
        scratch_shapes=[pltpu.VMEM((B, A), f32),
                        pltpu.VMEM((B, A), f32)],
        compiler_params=pltpu.CompilerParams(
            dimension_semantics=("arbitrary",)),
    )(actions, states, ww.astype(f32), bw.astype(f32),
      wb.astype(f32), bb.astype(f32))
    return out
